# baseline (speedup 1.0000x reference)
import functools
import jax
import jax.numpy as jnp
from jax import lax
from jax.experimental import pallas as pl
from jax.experimental.pallas import tpu as pltpu
from jax.experimental.pallas import tpu_sc as plsc

LANES = 16
NW = 32
CH = 256


def _sc_body(x_hbm, y_hbm, pk_hbm, xbuf, ybuf):
    wid = lax.axis_index("s") * 2 + lax.axis_index("c")
    rows = x_hbm.shape[0]
    per_w = rows // NW
    nchunks = per_w // CH

    def chunk_body(c, carry):
        base = wid * per_w + c * CH
        pltpu.sync_copy(x_hbm.at[pl.ds(base, CH)], xbuf)
        pltpu.sync_copy(xbuf, pk_hbm.at[pl.ds(base, CH)])

        def row_body(r, carry2):
            for j in range(128 // LANES):
                s = pl.ds(j * LANES, LANES)
                ybuf[r, s] = xbuf[r, s] + 1.0
            return carry2

        lax.fori_loop(0, CH, row_body, 0)
        pltpu.sync_copy(ybuf, y_hbm.at[pl.ds(base, CH)])
        return carry

    lax.fori_loop(0, nchunks, chunk_body, 0)


def kernel(key_states, token_idx, param):
    S, BS, D0, D1 = key_states.shape
    xt = jnp.swapaxes(key_states, 2, 3)
    x2 = xt.reshape(S * BS * D1, D0)
    mesh = plsc.VectorSubcoreMesh(core_axis_name="c", subcore_axis_name="s")
    run = pl.kernel(
        _sc_body,
        mesh=mesh,
        out_type=(
            jax.ShapeDtypeStruct(x2.shape, jnp.float32),
            jax.ShapeDtypeStruct(x2.shape, jnp.float32),
        ),
        scratch_types=[
            pltpu.VMEM((CH, D0), jnp.float32),
            pltpu.VMEM((CH, D0), jnp.float32),
        ],
    )
    y2, pk2 = run(x2)
    y = jnp.swapaxes(y2.reshape(S, BS, D1, D0), 2, 3)
    pk = jnp.swapaxes(pk2.reshape(S, BS, D1, D0), 2, 3)
    return y, pk


# speedup vs baseline: 1.0802x; 1.0802x over previous
import jax
import jax.numpy as jnp
from jax import lax
from jax.experimental import pallas as pl
from jax.experimental.pallas import tpu as pltpu
from jax.experimental.pallas import tpu_sc as plsc

NW = 32
CH = 512


def _tc_body(x_ref, y_ref):
    y_ref[...] = x_ref[...] + 1.0


def _sc_copy_body(x_hbm, pk_hbm, buf):
    wid = lax.axis_index("s") * 2 + lax.axis_index("c")
    rows = x_hbm.shape[0]
    per_w = rows // NW
    nchunks = per_w // CH

    def chunk_body(c, carry):
        base = wid * per_w + c * CH
        pltpu.sync_copy(x_hbm.at[pl.ds(base, CH)], buf)
        pltpu.sync_copy(buf, pk_hbm.at[pl.ds(base, CH)])
        return carry

    lax.fori_loop(0, nchunks, chunk_body, 0)


def kernel(key_states, token_idx, param):
    S, BS, D0, D1 = key_states.shape
    xt = jnp.swapaxes(key_states, 2, 3)
    x2 = xt.reshape(S * BS * D1, D0)

    mesh = plsc.VectorSubcoreMesh(core_axis_name="c", subcore_axis_name="s")
    pk2 = pl.kernel(
        _sc_copy_body,
        mesh=mesh,
        out_type=jax.ShapeDtypeStruct(x2.shape, jnp.float32),
        scratch_types=[pltpu.VMEM((CH, D0), jnp.float32)],
    )(x2)

    R = 64
    spec = pl.BlockSpec((R, BS, D1, D0), lambda i: (i, 0, 0, 0))
    y = pl.pallas_call(
        _tc_body,
        grid=(S // R,),
        in_specs=[spec],
        out_specs=spec,
        out_shape=jax.ShapeDtypeStruct((S, BS, D1, D0), key_states.dtype),
    )(xt)

    pk = jnp.swapaxes(pk2.reshape(S, BS, D1, D0), 2, 3)
    return jnp.swapaxes(y, 2, 3), pk


# speedup vs baseline: 1.6180x; 1.4978x over previous
import jax
import jax.numpy as jnp
from jax.experimental import pallas as pl


def _add_one_body(x_ref, y_ref, pk_ref):
    y_ref[...] = x_ref[...] + 1.0
    pk_ref[...] = x_ref[...]


def kernel(key_states, token_idx, param):
    S, BS, D0, D1 = key_states.shape
    xt = jnp.swapaxes(key_states, 2, 3)
    R = 64
    spec = pl.BlockSpec((R, BS, D1, D0), lambda i: (i, 0, 0, 0))
    y, pk = pl.pallas_call(
        _add_one_body,
        grid=(S // R,),
        in_specs=[spec],
        out_specs=[spec, spec],
        out_shape=[
            jax.ShapeDtypeStruct((S, BS, D1, D0), key_states.dtype),
            jax.ShapeDtypeStruct((S, BS, D1, D0), key_states.dtype),
        ],
    )(xt)
    return jnp.swapaxes(y, 2, 3), jnp.swapaxes(pk, 2, 3)
